# baseline (speedup 1.0000x reference)
import functools
import jax
import jax.numpy as jnp
from jax import lax
from jax.experimental import pallas as pl
from jax.experimental.pallas import tpu as pltpu
from jax.experimental.pallas import tpu_sc as plsc

D = 1024
E = 8
K = 2
I = 2048
NL = 4
T = 4096
A = T * K

BT = 256
NB = A // BT + NL + 1
S = NB * BT
S_TOT = 9728
TRASH = S

TB_R = 1024
TI = 512
IT = I // TI

NSUB = 16
NA_PER = A // NSUB
INIT_PER = S_TOT // NSUB

NW = 32
GCH = 64
NGCH = S // GCH
TPW = T // NW
CCH = 32


def _router_body(x_ref, gw_ref, ids_ref, w_ref, aux_ref, usage_acc, prob_acc):
    step = pl.program_id(0)
    nsteps = pl.num_programs(0)
    logits = lax.dot_general(x_ref[...], gw_ref[...], (((1,), (1,)), ((), ())),
                             preferred_element_type=jnp.float32)
    m = jnp.max(logits, axis=1, keepdims=True)
    p = jnp.exp(logits - m)
    probs = p / jnp.sum(p, axis=1, keepdims=True)

    top1v = jnp.max(probs, axis=1)
    top1i = jnp.argmax(probs, axis=1).astype(jnp.int32)
    cols = lax.broadcasted_iota(jnp.int32, probs.shape, 1)
    masked = jnp.where(cols == top1i[:, None], -jnp.inf, probs)
    top2v = jnp.max(masked, axis=1)
    top2i = jnp.argmax(masked, axis=1).astype(jnp.int32)

    denom = top1v + top2v + 1e-9
    ids_ref[...] = jnp.stack([top1i, top2i], axis=1)
    w_ref[...] = jnp.stack([top1v / denom, top2v / denom], axis=1)

    usage = jnp.sum((cols == top1i[:, None]).astype(jnp.float32), axis=0,
                    keepdims=True)
    psum = jnp.sum(probs, axis=0, keepdims=True)

    @pl.when(step == 0)
    def _():
        usage_acc[...] = jnp.zeros_like(usage_acc)
        prob_acc[...] = jnp.zeros_like(prob_acc)

    usage_acc[...] += usage
    prob_acc[...] += psum

    @pl.when(step == nsteps - 1)
    def _():
        aux_ref[...] = jnp.reshape(
            E * jnp.sum((usage_acc[...] / T) * (prob_acc[...] / T)), (1, 1))


def _router(x, gate_w):
    return pl.pallas_call(
        _router_body,
        grid=(T // TB_R,),
        in_specs=[
            pl.BlockSpec((TB_R, D), lambda i: (i, 0)),
            pl.BlockSpec((E, D), lambda i: (0, 0)),
        ],
        out_specs=[
            pl.BlockSpec((TB_R, K), lambda i: (i, 0)),
            pl.BlockSpec((TB_R, K), lambda i: (i, 0)),
            pl.BlockSpec((1, 1), lambda i: (0, 0)),
        ],
        out_shape=[
            jax.ShapeDtypeStruct((T, K), jnp.int32),
            jax.ShapeDtypeStruct((T, K), jnp.float32),
            jax.ShapeDtypeStruct((1, 1), jnp.float32),
        ],
        scratch_shapes=[
            pltpu.VMEM((1, E), jnp.float32),
            pltpu.VMEM((1, E), jnp.float32),
        ],
    )(x, gate_w)


def _dispatch_body(ids_hbm, w_hbm,
                   stok_hbm, sw_hbm, inv_hbm, be_hbm, na_hbm,
                   ids_v, w_v, tok_v, inv_v, pos_b, mv, bev,
                   zi, zf):
    wid = lax.axis_index("s")
    base = wid * NA_PER
    iota = lax.iota(jnp.int32, 16)
    zv = jnp.zeros((16,), jnp.int32)
    ones = zv + 1
    evec = [zv + e for e in range(NL)]

    pltpu.sync_copy(ids_hbm, ids_v)
    pltpu.sync_copy(w_hbm.at[pl.ds(base, NA_PER)], w_v)

    myvreg = wid * (NA_PER // 16)

    def cbody(i, c):
        v = ids_v[pl.ds(i * 16, 16)]
        pre = jnp.where(i < myvreg, jnp.int32(1), jnp.int32(0))
        out = []
        for e in range(NL):
            cnt = jnp.sum(jnp.where(v == evec[e], ones, zv))
            out.append(c[2 * e] + cnt)
            out.append(c[2 * e + 1] + cnt * pre)
        return tuple(out)

    z = jnp.int32(0)
    acc = lax.fori_loop(0, A // 16, cbody, (z,) * (2 * NL))
    tot = [acc[2 * e] for e in range(NL)]
    excl = [acc[2 * e + 1] for e in range(NL)]

    nb = [(tot[e] + (BT - 1)) // BT for e in range(NL)]
    sb = [jnp.int32(1)]
    for e in range(NL):
        sb.append(sb[e] + nb[e])
    na_blocks = sb[NL]
    offs0 = [sb[e] * BT + excl[e] for e in range(NL)]

    mv[...] = zv + na_blocks
    sbv = [zv + sb[e] for e in (1, 2, 3)]
    for j in range(3):
        bv = zv + (j * 16) + iota
        val = (jnp.where(bv >= sbv[0], ones, zv)
               + jnp.where(bv >= sbv[1], ones, zv)
               + jnp.where(bv >= sbv[2], ones, zv))
        bev[pl.ds(j * 16, 16)] = val
    pltpu.sync_copy(bev, be_hbm)
    pltpu.sync_copy(mv, na_hbm)

    for j in range(INIT_PER // 16):
        zi[pl.ds(j * 16, 16)] = jnp.zeros((16,), jnp.int32)
        zf[pl.ds(j * 16, 16)] = jnp.zeros((16,), jnp.float32)
    ibase = wid * INIT_PER
    pltpu.sync_copy(zi, stok_hbm.at[pl.ds(ibase, INIT_PER)])
    pltpu.sync_copy(zf, sw_hbm.at[pl.ds(ibase, INIT_PER)])

    plsc.subcore_barrier()

    nlv = zv + NL
    trashv = zv + TRASH

    def sbody(i, offs):
        v = ids_v[pl.ds(base + i * 16, 16)]
        a_vec = (zv + base) + i * 16 + iota
        tok = lax.shift_right_logical(a_vec, 1)
        pos = trashv
        new_offs = []
        for e in range(NL):
            m = v == evec[e]
            mi = jnp.where(m, ones, zv)
            ranks = plsc.cumsum(mi)
            pos = jnp.where(m, ranks + (offs[e] - 1), pos)
            new_offs.append(offs[e] + jnp.sum(mi))
        inv16 = jnp.where(v < nlv, pos, zv)
        tok_v[pl.ds(i * 16, 16)] = tok
        inv_v[pl.ds(i * 16, 16)] = inv16
        r = i // 8
        col = (i % 8) * 16
        pos_b[r, pl.ds(col, 16)] = pos
        return tuple(new_offs)

    lax.fori_loop(0, NA_PER // 16, sbody, tuple(offs0))

    pltpu.sync_copy(inv_v, inv_hbm.at[pl.ds(base, NA_PER)])
    for j in range(NA_PER // 128):
        pltpu.sync_copy(tok_v.at[pl.ds(j * 128, 128)],
                        stok_hbm.at[pos_b.at[j]])
        pltpu.sync_copy(w_v.at[pl.ds(j * 128, 128)],
                        sw_hbm.at[pos_b.at[j]])


def _dispatch(ids_flat, w_flat):
    mesh = plsc.VectorSubcoreMesh(core_axis_name="c", subcore_axis_name="s",
                                  num_cores=1)
    f = functools.partial(
        pl.kernel,
        mesh=mesh,
        compiler_params=pltpu.CompilerParams(needs_layout_passes=False),
        out_type=[
            jax.ShapeDtypeStruct((S_TOT,), jnp.int32),
            jax.ShapeDtypeStruct((S_TOT,), jnp.float32),
            jax.ShapeDtypeStruct((A,), jnp.int32),
            jax.ShapeDtypeStruct((48,), jnp.int32),
            jax.ShapeDtypeStruct((16,), jnp.int32),
        ],
        scratch_types=[
            pltpu.VMEM((A,), jnp.int32),
            pltpu.VMEM((NA_PER,), jnp.float32),
            pltpu.VMEM((NA_PER,), jnp.int32),
            pltpu.VMEM((NA_PER,), jnp.int32),
            pltpu.VMEM((NA_PER // 128, 128), jnp.int32),
            pltpu.VMEM((16,), jnp.int32),
            pltpu.VMEM((48,), jnp.int32),
            pltpu.VMEM((INIT_PER,), jnp.int32),
            pltpu.VMEM((INIT_PER,), jnp.float32),
        ],
    )(_dispatch_body)
    return f(ids_flat, w_flat)


def _gather_body(stok_hbm, x_hbm, na_hbm, xs_hbm, idx_v, rows_v, nav, sem):
    w = lax.axis_index("s") * 2 + lax.axis_index("c")
    pltpu.sync_copy(na_hbm, nav)
    n_rows = nav[...][0] * BT
    for j in range(NGCH // NW + 1):
        cid = w + j * NW
        base = cid * GCH

        @pl.when((cid < NGCH) & (base < n_rows))
        def _():
            pltpu.sync_copy(stok_hbm.at[pl.ds(base, GCH)], idx_v)
            pltpu.async_copy(x_hbm.at[idx_v], rows_v, sem).wait()
            pltpu.sync_copy(rows_v, xs_hbm.at[pl.ds(base, GCH)])


def _gather(stok, x, na):
    mesh = plsc.VectorSubcoreMesh(core_axis_name="c", subcore_axis_name="s")
    f = functools.partial(
        pl.kernel,
        mesh=mesh,
        compiler_params=pltpu.CompilerParams(needs_layout_passes=False),
        out_type=jax.ShapeDtypeStruct((S, D), jnp.float32),
        scratch_types=[
            pltpu.VMEM((GCH,), jnp.int32),
            pltpu.VMEM((GCH, D), jnp.float32),
            pltpu.VMEM((16,), jnp.int32),
            pltpu.SemaphoreType.DMA,
        ],
    )(_gather_body)
    return f(stok, x, na)


def _gmm_body(be_ref, na_ref, sw_ref, xs_ref, eg_ref, eu_ref, ed_ref,
              out_ref, acc):
    b = pl.program_id(0)
    it = pl.program_id(1)
    na = na_ref[0]

    @pl.when(b < na)
    def _():
        @pl.when(it == 0)
        def _():
            acc[...] = jnp.zeros_like(acc)

        x = xs_ref[...]
        g = lax.dot_general(x, eg_ref[0], (((1,), (1,)), ((), ())),
                            preferred_element_type=jnp.float32)
        u = lax.dot_general(x, eu_ref[0], (((1,), (1,)), ((), ())),
                            preferred_element_type=jnp.float32)
        h = g * jax.nn.sigmoid(g) * u
        acc[...] += lax.dot_general(h, ed_ref[0], (((1,), (1,)), ((), ())),
                                    preferred_element_type=jnp.float32)

        @pl.when(it == IT - 1)
        def _():
            out_ref[...] = acc[...] * sw_ref[0, 0, :][:, None]


def _gmm(be, na, sw3, xs, eg, eu, ed):
    def bclamp(b, na_ref):
        return jnp.minimum(b, na_ref[0] - 1)

    grid_spec = pltpu.PrefetchScalarGridSpec(
        num_scalar_prefetch=2,
        grid=(NB, IT),
        in_specs=[
            pl.BlockSpec((1, 1, BT),
                         lambda b, it, be_r, na_r: (bclamp(b, na_r), 0, 0)),
            pl.BlockSpec((BT, D),
                         lambda b, it, be_r, na_r: (bclamp(b, na_r), 0)),
            pl.BlockSpec((1, TI, D),
                         lambda b, it, be_r, na_r:
                         (be_r[bclamp(b, na_r)],
                          jnp.where(b < na_r[0], it, IT - 1), 0)),
            pl.BlockSpec((1, TI, D),
                         lambda b, it, be_r, na_r:
                         (be_r[bclamp(b, na_r)],
                          jnp.where(b < na_r[0], it, IT - 1), 0)),
            pl.BlockSpec((1, D, TI),
                         lambda b, it, be_r, na_r:
                         (be_r[bclamp(b, na_r)], 0,
                          jnp.where(b < na_r[0], it, IT - 1))),
        ],
        out_specs=pl.BlockSpec((BT, D), lambda b, it, be_r, na_r: (b, 0)),
        scratch_shapes=[pltpu.VMEM((BT, D), jnp.float32)],
    )
    return pl.pallas_call(
        _gmm_body,
        grid_spec=grid_spec,
        out_shape=jax.ShapeDtypeStruct((S, D), jnp.float32),
    )(be, na, sw3, xs, eg, eu, ed)


def _combine_body(inv_hbm, ys_hbm, out_hbm, iv64, idx0, idx1, r0, r1, sem):
    w = lax.axis_index("s") * 2 + lax.axis_index("c")
    iota = lax.iota(jnp.int32, 16)
    for c in range(TPW // CCH):
        tbase = w * TPW + c * CCH
        pltpu.sync_copy(inv_hbm.at[pl.ds(2 * tbase, 2 * CCH)], iv64)
        for r in range(CCH // 16):
            lanes = (r * 16 + iota) * 2
            idx0[pl.ds(r * 16, 16)] = plsc.load_gather(iv64, [lanes])
            idx1[pl.ds(r * 16, 16)] = plsc.load_gather(iv64, [lanes + 1])
        pltpu.async_copy(ys_hbm.at[idx0], r0, sem).wait()
        pltpu.async_copy(ys_hbm.at[idx1], r1, sem).wait()

        def orow(rr, _):
            def ocol(cc, _):
                sl = pl.ds(cc * 16, 16)
                r0[rr, sl] = r0[rr, sl] + r1[rr, sl]
                return 0
            return lax.fori_loop(0, D // 16, ocol, 0)

        lax.fori_loop(0, CCH, orow, 0)
        pltpu.sync_copy(r0, out_hbm.at[pl.ds(tbase, CCH)])


def _combine(inv, ys):
    mesh = plsc.VectorSubcoreMesh(core_axis_name="c", subcore_axis_name="s")
    f = functools.partial(
        pl.kernel,
        mesh=mesh,
        compiler_params=pltpu.CompilerParams(needs_layout_passes=False),
        out_type=jax.ShapeDtypeStruct((T, D), jnp.float32),
        scratch_types=[
            pltpu.VMEM((2 * CCH,), jnp.int32),
            pltpu.VMEM((CCH,), jnp.int32),
            pltpu.VMEM((CCH,), jnp.int32),
            pltpu.VMEM((CCH, D), jnp.float32),
            pltpu.VMEM((CCH, D), jnp.float32),
            pltpu.SemaphoreType.DMA,
        ],
    )(_combine_body)
    return f(inv, ys)


def kernel(x, gate_w, expert_gate, expert_up, expert_down):
    x_flat = x.reshape(-1, x.shape[-1])
    topk_ids, topk_w, aux = _router(x_flat, gate_w)
    stok, sw, inv, be, na = _dispatch(topk_ids.reshape(A),
                                      topk_w.reshape(A))
    xs = _gather(stok, x_flat, na)
    sw3 = sw[:S].reshape(NB, 1, BT)
    ys = _gmm(be, na, sw3, xs, expert_gate, expert_up, expert_down)
    out = _combine(inv, ys)
    return out.reshape(x.shape), aux[0, 0]


# speedup vs baseline: 1.0047x; 1.0047x over previous
import functools
import jax
import jax.numpy as jnp
from jax import lax
from jax.experimental import pallas as pl
from jax.experimental.pallas import tpu as pltpu
from jax.experimental.pallas import tpu_sc as plsc

D = 1024
E = 8
K = 2
I = 2048
NL = 4
T = 4096
A = T * K

BT = 256
NB = A // BT + NL + 1
S = NB * BT
S_TOT = 9728
TRASH = S

TB_R = 1024
TI = 512
IT = I // TI

NSUB = 16
NA_PER = A // NSUB
INIT_PER = S_TOT // NSUB

NW = 32
GCH = 64
NGCH = S // GCH
TPW = T // NW
CCH = 32


def _router_body(x_ref, gw_ref, ids_ref, w_ref, aux_ref, usage_acc, prob_acc):
    step = pl.program_id(0)
    nsteps = pl.num_programs(0)
    logits = lax.dot_general(x_ref[...], gw_ref[...], (((1,), (1,)), ((), ())),
                             preferred_element_type=jnp.float32)
    m = jnp.max(logits, axis=1, keepdims=True)
    p = jnp.exp(logits - m)
    probs = p / jnp.sum(p, axis=1, keepdims=True)

    top1v = jnp.max(probs, axis=1)
    top1i = jnp.argmax(probs, axis=1).astype(jnp.int32)
    cols = lax.broadcasted_iota(jnp.int32, probs.shape, 1)
    masked = jnp.where(cols == top1i[:, None], -jnp.inf, probs)
    top2v = jnp.max(masked, axis=1)
    top2i = jnp.argmax(masked, axis=1).astype(jnp.int32)

    denom = top1v + top2v + 1e-9
    ids_ref[...] = jnp.stack([top1i, top2i], axis=1)
    w_ref[...] = jnp.stack([top1v / denom, top2v / denom], axis=1)

    usage = jnp.sum((cols == top1i[:, None]).astype(jnp.float32), axis=0,
                    keepdims=True)
    psum = jnp.sum(probs, axis=0, keepdims=True)

    @pl.when(step == 0)
    def _():
        usage_acc[...] = jnp.zeros_like(usage_acc)
        prob_acc[...] = jnp.zeros_like(prob_acc)

    usage_acc[...] += usage
    prob_acc[...] += psum

    @pl.when(step == nsteps - 1)
    def _():
        aux_ref[...] = jnp.reshape(
            E * jnp.sum((usage_acc[...] / T) * (prob_acc[...] / T)), (1, 1))


def _router(x, gate_w):
    return pl.pallas_call(
        _router_body,
        grid=(T // TB_R,),
        in_specs=[
            pl.BlockSpec((TB_R, D), lambda i: (i, 0)),
            pl.BlockSpec((E, D), lambda i: (0, 0)),
        ],
        out_specs=[
            pl.BlockSpec((TB_R, K), lambda i: (i, 0)),
            pl.BlockSpec((TB_R, K), lambda i: (i, 0)),
            pl.BlockSpec((1, 1), lambda i: (0, 0)),
        ],
        out_shape=[
            jax.ShapeDtypeStruct((T, K), jnp.int32),
            jax.ShapeDtypeStruct((T, K), jnp.float32),
            jax.ShapeDtypeStruct((1, 1), jnp.float32),
        ],
        scratch_shapes=[
            pltpu.VMEM((1, E), jnp.float32),
            pltpu.VMEM((1, E), jnp.float32),
        ],
    )(x, gate_w)


def _dispatch_body(ids_hbm, w_hbm,
                   stok_hbm, sw_hbm, inv_hbm, be_hbm, na_hbm,
                   ids_v, w_v, tok_v, inv_v, pos_b, mv, bev,
                   zi, zf):
    wid = lax.axis_index("s")
    base = wid * NA_PER
    iota = lax.iota(jnp.int32, 16)
    zv = jnp.zeros((16,), jnp.int32)
    ones = zv + 1
    evec = [zv + e for e in range(NL)]

    pltpu.sync_copy(ids_hbm, ids_v)
    pltpu.sync_copy(w_hbm.at[pl.ds(base, NA_PER)], w_v)

    myvreg = wid * (NA_PER // 16)

    def cbody(i, c):
        v = ids_v[pl.ds(i * 16, 16)]
        prevec = zv + jnp.where(i < myvreg, jnp.int32(1), jnp.int32(0))
        out = []
        for e in range(NL):
            cnt = jnp.where(v == evec[e], ones, zv)
            out.append(c[2 * e] + cnt)
            out.append(c[2 * e + 1] + cnt * prevec)
        return tuple(out)

    acc = lax.fori_loop(0, A // 16, cbody, (zv,) * (2 * NL))
    tot = [jnp.sum(acc[2 * e]) for e in range(NL)]
    excl = [jnp.sum(acc[2 * e + 1]) for e in range(NL)]

    nb = [(tot[e] + (BT - 1)) // BT for e in range(NL)]
    sb = [jnp.int32(1)]
    for e in range(NL):
        sb.append(sb[e] + nb[e])
    na_blocks = sb[NL]
    offs0 = [sb[e] * BT + excl[e] for e in range(NL)]

    mv[...] = zv + na_blocks
    sbv = [zv + sb[e] for e in (1, 2, 3)]
    for j in range(3):
        bv = zv + (j * 16) + iota
        val = (jnp.where(bv >= sbv[0], ones, zv)
               + jnp.where(bv >= sbv[1], ones, zv)
               + jnp.where(bv >= sbv[2], ones, zv))
        bev[pl.ds(j * 16, 16)] = val
    pltpu.sync_copy(bev, be_hbm)
    pltpu.sync_copy(mv, na_hbm)

    for j in range(INIT_PER // 16):
        zi[pl.ds(j * 16, 16)] = jnp.zeros((16,), jnp.int32)
        zf[pl.ds(j * 16, 16)] = jnp.zeros((16,), jnp.float32)
    ibase = wid * INIT_PER
    pltpu.sync_copy(zi, stok_hbm.at[pl.ds(ibase, INIT_PER)])
    pltpu.sync_copy(zf, sw_hbm.at[pl.ds(ibase, INIT_PER)])

    plsc.subcore_barrier()

    nlv = zv + NL
    trashv = zv + TRASH

    def sbody(i, offs):
        v = ids_v[pl.ds(base + i * 16, 16)]
        a_vec = (zv + base) + i * 16 + iota
        tok = lax.shift_right_logical(a_vec, 1)
        pos = trashv
        new_offs = []
        for e in range(NL):
            m = v == evec[e]
            mi = jnp.where(m, ones, zv)
            ranks = plsc.cumsum(mi)
            pos = jnp.where(m, ranks + (offs[e] - 1), pos)
            new_offs.append(offs[e] + ranks[15])
        inv16 = jnp.where(v < nlv, pos, zv)
        tok_v[pl.ds(i * 16, 16)] = tok
        inv_v[pl.ds(i * 16, 16)] = inv16
        r = i // 8
        col = (i % 8) * 16
        pos_b[r, pl.ds(col, 16)] = pos
        return tuple(new_offs)

    lax.fori_loop(0, NA_PER // 16, sbody, tuple(offs0))

    pltpu.sync_copy(inv_v, inv_hbm.at[pl.ds(base, NA_PER)])
    for j in range(NA_PER // 128):
        pltpu.sync_copy(tok_v.at[pl.ds(j * 128, 128)],
                        stok_hbm.at[pos_b.at[j]])
        pltpu.sync_copy(w_v.at[pl.ds(j * 128, 128)],
                        sw_hbm.at[pos_b.at[j]])


def _dispatch(ids_flat, w_flat):
    mesh = plsc.VectorSubcoreMesh(core_axis_name="c", subcore_axis_name="s",
                                  num_cores=1)
    f = functools.partial(
        pl.kernel,
        mesh=mesh,
        compiler_params=pltpu.CompilerParams(needs_layout_passes=False),
        out_type=[
            jax.ShapeDtypeStruct((S_TOT,), jnp.int32),
            jax.ShapeDtypeStruct((S_TOT,), jnp.float32),
            jax.ShapeDtypeStruct((A,), jnp.int32),
            jax.ShapeDtypeStruct((48,), jnp.int32),
            jax.ShapeDtypeStruct((16,), jnp.int32),
        ],
        scratch_types=[
            pltpu.VMEM((A,), jnp.int32),
            pltpu.VMEM((NA_PER,), jnp.float32),
            pltpu.VMEM((NA_PER,), jnp.int32),
            pltpu.VMEM((NA_PER,), jnp.int32),
            pltpu.VMEM((NA_PER // 128, 128), jnp.int32),
            pltpu.VMEM((16,), jnp.int32),
            pltpu.VMEM((48,), jnp.int32),
            pltpu.VMEM((INIT_PER,), jnp.int32),
            pltpu.VMEM((INIT_PER,), jnp.float32),
        ],
    )(_dispatch_body)
    return f(ids_flat, w_flat)


def _gather_body(stok_hbm, x_hbm, na_hbm, xs_hbm, idx_v, rows_v, nav, sem):
    w = lax.axis_index("s") * 2 + lax.axis_index("c")
    pltpu.sync_copy(na_hbm, nav)
    n_rows = nav[...][0] * BT
    for j in range(NGCH // NW + 1):
        cid = w + j * NW
        base = cid * GCH

        @pl.when((cid < NGCH) & (base < n_rows))
        def _():
            pltpu.sync_copy(stok_hbm.at[pl.ds(base, GCH)], idx_v)
            pltpu.async_copy(x_hbm.at[idx_v], rows_v, sem).wait()
            pltpu.sync_copy(rows_v, xs_hbm.at[pl.ds(base, GCH)])


def _gather(stok, x, na):
    mesh = plsc.VectorSubcoreMesh(core_axis_name="c", subcore_axis_name="s")
    f = functools.partial(
        pl.kernel,
        mesh=mesh,
        compiler_params=pltpu.CompilerParams(needs_layout_passes=False),
        out_type=jax.ShapeDtypeStruct((S, D), jnp.float32),
        scratch_types=[
            pltpu.VMEM((GCH,), jnp.int32),
            pltpu.VMEM((GCH, D), jnp.float32),
            pltpu.VMEM((16,), jnp.int32),
            pltpu.SemaphoreType.DMA,
        ],
    )(_gather_body)
    return f(stok, x, na)


def _gmm_body(be_ref, na_ref, sw_ref, xs_ref, eg_ref, eu_ref, ed_ref,
              out_ref, acc):
    b = pl.program_id(0)
    it = pl.program_id(1)
    na = na_ref[0]

    @pl.when(b < na)
    def _():
        @pl.when(it == 0)
        def _():
            acc[...] = jnp.zeros_like(acc)

        x = xs_ref[...]
        g = lax.dot_general(x, eg_ref[0], (((1,), (1,)), ((), ())),
                            preferred_element_type=jnp.float32)
        u = lax.dot_general(x, eu_ref[0], (((1,), (1,)), ((), ())),
                            preferred_element_type=jnp.float32)
        h = g * jax.nn.sigmoid(g) * u
        acc[...] += lax.dot_general(h, ed_ref[0], (((1,), (1,)), ((), ())),
                                    preferred_element_type=jnp.float32)

        @pl.when(it == IT - 1)
        def _():
            out_ref[...] = acc[...] * sw_ref[0, 0, :][:, None]


def _gmm(be, na, sw3, xs, eg, eu, ed):
    def bclamp(b, na_ref):
        return jnp.minimum(b, na_ref[0] - 1)

    grid_spec = pltpu.PrefetchScalarGridSpec(
        num_scalar_prefetch=2,
        grid=(NB, IT),
        in_specs=[
            pl.BlockSpec((1, 1, BT),
                         lambda b, it, be_r, na_r: (bclamp(b, na_r), 0, 0)),
            pl.BlockSpec((BT, D),
                         lambda b, it, be_r, na_r: (bclamp(b, na_r), 0)),
            pl.BlockSpec((1, TI, D),
                         lambda b, it, be_r, na_r:
                         (be_r[bclamp(b, na_r)],
                          jnp.where(b < na_r[0], it, IT - 1), 0)),
            pl.BlockSpec((1, TI, D),
                         lambda b, it, be_r, na_r:
                         (be_r[bclamp(b, na_r)],
                          jnp.where(b < na_r[0], it, IT - 1), 0)),
            pl.BlockSpec((1, D, TI),
                         lambda b, it, be_r, na_r:
                         (be_r[bclamp(b, na_r)], 0,
                          jnp.where(b < na_r[0], it, IT - 1))),
        ],
        out_specs=pl.BlockSpec((BT, D), lambda b, it, be_r, na_r: (b, 0)),
        scratch_shapes=[pltpu.VMEM((BT, D), jnp.float32)],
    )
    return pl.pallas_call(
        _gmm_body,
        grid_spec=grid_spec,
        out_shape=jax.ShapeDtypeStruct((S, D), jnp.float32),
    )(be, na, sw3, xs, eg, eu, ed)


def _combine_body(inv_hbm, ys_hbm, out_hbm, iv64, idx0, idx1, r0, r1, sem):
    w = lax.axis_index("s") * 2 + lax.axis_index("c")
    iota = lax.iota(jnp.int32, 16)
    for c in range(TPW // CCH):
        tbase = w * TPW + c * CCH
        pltpu.sync_copy(inv_hbm.at[pl.ds(2 * tbase, 2 * CCH)], iv64)
        for r in range(CCH // 16):
            lanes = (r * 16 + iota) * 2
            idx0[pl.ds(r * 16, 16)] = plsc.load_gather(iv64, [lanes])
            idx1[pl.ds(r * 16, 16)] = plsc.load_gather(iv64, [lanes + 1])
        pltpu.async_copy(ys_hbm.at[idx0], r0, sem).wait()
        pltpu.async_copy(ys_hbm.at[idx1], r1, sem).wait()

        def orow(rr, _):
            for cc in range(D // 16):
                sl = pl.ds(cc * 16, 16)
                r0[rr, sl] = r0[rr, sl] + r1[rr, sl]
            return 0

        lax.fori_loop(0, CCH, orow, 0)
        pltpu.sync_copy(r0, out_hbm.at[pl.ds(tbase, CCH)])


def _combine(inv, ys):
    mesh = plsc.VectorSubcoreMesh(core_axis_name="c", subcore_axis_name="s")
    f = functools.partial(
        pl.kernel,
        mesh=mesh,
        compiler_params=pltpu.CompilerParams(needs_layout_passes=False),
        out_type=jax.ShapeDtypeStruct((T, D), jnp.float32),
        scratch_types=[
            pltpu.VMEM((2 * CCH,), jnp.int32),
            pltpu.VMEM((CCH,), jnp.int32),
            pltpu.VMEM((CCH,), jnp.int32),
            pltpu.VMEM((CCH, D), jnp.float32),
            pltpu.VMEM((CCH, D), jnp.float32),
            pltpu.SemaphoreType.DMA,
        ],
    )(_combine_body)
    return f(inv, ys)


def kernel(x, gate_w, expert_gate, expert_up, expert_down):
    x_flat = x.reshape(-1, x.shape[-1])
    topk_ids, topk_w, aux = _router(x_flat, gate_w)
    stok, sw, inv, be, na = _dispatch(topk_ids.reshape(A),
                                      topk_w.reshape(A))
    xs = _gather(stok, x_flat, na)
    sw3 = sw[:S].reshape(NB, 1, BT)
    ys = _gmm(be, na, sw3, xs, expert_gate, expert_up, expert_down)
    out = _combine(inv, ys)
    return out.reshape(x.shape), aux[0, 0]


# speedup vs baseline: 2.3157x; 2.3048x over previous
import functools
import jax
import jax.numpy as jnp
from jax import lax
from jax.experimental import pallas as pl
from jax.experimental.pallas import tpu as pltpu
from jax.experimental.pallas import tpu_sc as plsc

D = 1024
E = 8
K = 2
I = 2048
NL = 4
T = 4096
A = T * K

BT = 256
NB = A // BT + NL + 1
S = NB * BT
S_TOT = 9728
TRASH = S

TB_R = 1024
TI = 512
IT = I // TI

NSUB = 16
NA_PER = A // NSUB
INIT_PER = S_TOT // NSUB

NW = 32
GCH = 64
NGCH = S // GCH
TPW = T // NW
CCH = 32


def _router_body(x_ref, gw_ref, ids_ref, w_ref, aux_ref, usage_acc, prob_acc):
    step = pl.program_id(0)
    nsteps = pl.num_programs(0)
    logits = lax.dot_general(x_ref[...], gw_ref[...], (((1,), (1,)), ((), ())),
                             preferred_element_type=jnp.float32)
    m = jnp.max(logits, axis=1, keepdims=True)
    p = jnp.exp(logits - m)
    probs = p / jnp.sum(p, axis=1, keepdims=True)

    top1v = jnp.max(probs, axis=1)
    top1i = jnp.argmax(probs, axis=1).astype(jnp.int32)
    cols = lax.broadcasted_iota(jnp.int32, probs.shape, 1)
    masked = jnp.where(cols == top1i[:, None], -jnp.inf, probs)
    top2v = jnp.max(masked, axis=1)
    top2i = jnp.argmax(masked, axis=1).astype(jnp.int32)

    denom = top1v + top2v + 1e-9
    ids_ref[...] = jnp.stack([top1i, top2i], axis=1)
    w_ref[...] = jnp.stack([top1v / denom, top2v / denom], axis=1)

    usage = jnp.sum((cols == top1i[:, None]).astype(jnp.float32), axis=0,
                    keepdims=True)
    psum = jnp.sum(probs, axis=0, keepdims=True)

    @pl.when(step == 0)
    def _():
        usage_acc[...] = jnp.zeros_like(usage_acc)
        prob_acc[...] = jnp.zeros_like(prob_acc)

    usage_acc[...] += usage
    prob_acc[...] += psum

    @pl.when(step == nsteps - 1)
    def _():
        aux_ref[...] = jnp.reshape(
            E * jnp.sum((usage_acc[...] / T) * (prob_acc[...] / T)), (1, 1))


def _router(x, gate_w):
    return pl.pallas_call(
        _router_body,
        grid=(T // TB_R,),
        in_specs=[
            pl.BlockSpec((TB_R, D), lambda i: (i, 0)),
            pl.BlockSpec((E, D), lambda i: (0, 0)),
        ],
        out_specs=[
            pl.BlockSpec((TB_R, K), lambda i: (i, 0)),
            pl.BlockSpec((TB_R, K), lambda i: (i, 0)),
            pl.BlockSpec((1, 1), lambda i: (0, 0)),
        ],
        out_shape=[
            jax.ShapeDtypeStruct((T, K), jnp.int32),
            jax.ShapeDtypeStruct((T, K), jnp.float32),
            jax.ShapeDtypeStruct((1, 1), jnp.float32),
        ],
        scratch_shapes=[
            pltpu.VMEM((1, E), jnp.float32),
            pltpu.VMEM((1, E), jnp.float32),
        ],
    )(x, gate_w)


def _dispatch_body(ids_hbm, w_hbm,
                   stok_hbm, sw_hbm, inv_hbm, be_hbm, na_hbm,
                   ids_v, w_v, tok_v, inv_v, pos_b, mv, bev,
                   zi, zf, stok_s, sw_s):
    wid = lax.axis_index("s")
    base = wid * NA_PER
    iota = lax.iota(jnp.int32, 16)
    zv = jnp.zeros((16,), jnp.int32)
    ones = zv + 1
    evec = [zv + e for e in range(NL)]

    pltpu.sync_copy(ids_hbm, ids_v)
    pltpu.sync_copy(w_hbm.at[pl.ds(base, NA_PER)], w_v)

    myvreg = wid * (NA_PER // 16)

    def cbody(i, c):
        v = ids_v[pl.ds(i * 16, 16)]
        prevec = zv + jnp.where(i < myvreg, jnp.int32(1), jnp.int32(0))
        out = []
        for e in range(NL):
            cnt = jnp.where(v == evec[e], ones, zv)
            out.append(c[2 * e] + cnt)
            out.append(c[2 * e + 1] + cnt * prevec)
        return tuple(out)

    acc = lax.fori_loop(0, A // 16, cbody, (zv,) * (2 * NL))
    tot = [jnp.sum(acc[2 * e]) for e in range(NL)]
    excl = [jnp.sum(acc[2 * e + 1]) for e in range(NL)]

    nb = [(tot[e] + (BT - 1)) // BT for e in range(NL)]
    sb = [jnp.int32(1)]
    for e in range(NL):
        sb.append(sb[e] + nb[e])
    na_blocks = sb[NL]
    offs0 = [sb[e] * BT + excl[e] for e in range(NL)]

    mv[...] = zv + na_blocks
    sbv = [zv + sb[e] for e in (1, 2, 3)]
    for j in range(3):
        bv = zv + (j * 16) + iota
        val = (jnp.where(bv >= sbv[0], ones, zv)
               + jnp.where(bv >= sbv[1], ones, zv)
               + jnp.where(bv >= sbv[2], ones, zv))
        bev[pl.ds(j * 16, 16)] = val
    pltpu.sync_copy(bev, be_hbm)
    pltpu.sync_copy(mv, na_hbm)

    for j in range(INIT_PER // 16):
        zi[pl.ds(j * 16, 16)] = jnp.zeros((16,), jnp.int32)
        zf[pl.ds(j * 16, 16)] = jnp.zeros((16,), jnp.float32)
    ibase = wid * INIT_PER
    pltpu.sync_copy(zi, stok_s.at[pl.ds(ibase, INIT_PER)])
    pltpu.sync_copy(zf, sw_s.at[pl.ds(ibase, INIT_PER)])

    plsc.subcore_barrier()

    nlv = zv + NL
    trashv = zv + TRASH

    def sbody(i, offs):
        v = ids_v[pl.ds(base + i * 16, 16)]
        a_vec = (zv + base) + i * 16 + iota
        tok = lax.shift_right_logical(a_vec, 1)
        pos = trashv
        new_offs = []
        for e in range(NL):
            m = v == evec[e]
            mi = jnp.where(m, ones, zv)
            ranks = plsc.cumsum(mi)
            pos = jnp.where(m, ranks + (offs[e] - 1), pos)
            new_offs.append(offs[e] + ranks[15])
        inv16 = jnp.where(v < nlv, pos, zv)
        tok_v[pl.ds(i * 16, 16)] = tok
        inv_v[pl.ds(i * 16, 16)] = inv16
        r = i // 8
        col = (i % 8) * 16
        pos_b[r, pl.ds(col, 16)] = pos
        return tuple(new_offs)

    lax.fori_loop(0, NA_PER // 16, sbody, tuple(offs0))

    pltpu.sync_copy(inv_v, inv_hbm.at[pl.ds(base, NA_PER)])
    for j in range(NA_PER // 128):
        pltpu.sync_copy(tok_v.at[pl.ds(j * 128, 128)],
                        stok_s.at[pos_b.at[j]])
        pltpu.sync_copy(w_v.at[pl.ds(j * 128, 128)],
                        sw_s.at[pos_b.at[j]])

    plsc.subcore_barrier()

    pltpu.sync_copy(stok_s.at[pl.ds(ibase, INIT_PER)], zi)
    pltpu.sync_copy(zi, stok_hbm.at[pl.ds(ibase, INIT_PER)])
    pltpu.sync_copy(sw_s.at[pl.ds(ibase, INIT_PER)], zf)
    pltpu.sync_copy(zf, sw_hbm.at[pl.ds(ibase, INIT_PER)])


def _dispatch(ids_flat, w_flat):
    mesh = plsc.VectorSubcoreMesh(core_axis_name="c", subcore_axis_name="s",
                                  num_cores=1)
    f = functools.partial(
        pl.kernel,
        mesh=mesh,
        compiler_params=pltpu.CompilerParams(needs_layout_passes=False),
        out_type=[
            jax.ShapeDtypeStruct((S_TOT,), jnp.int32),
            jax.ShapeDtypeStruct((S_TOT,), jnp.float32),
            jax.ShapeDtypeStruct((A,), jnp.int32),
            jax.ShapeDtypeStruct((48,), jnp.int32),
            jax.ShapeDtypeStruct((16,), jnp.int32),
        ],
        scratch_types=[
            pltpu.VMEM((A,), jnp.int32),
            pltpu.VMEM((NA_PER,), jnp.float32),
            pltpu.VMEM((NA_PER,), jnp.int32),
            pltpu.VMEM((NA_PER,), jnp.int32),
            pltpu.VMEM((NA_PER // 128, 128), jnp.int32),
            pltpu.VMEM((16,), jnp.int32),
            pltpu.VMEM((48,), jnp.int32),
            pltpu.VMEM((INIT_PER,), jnp.int32),
            pltpu.VMEM((INIT_PER,), jnp.float32),
            pltpu.VMEM_SHARED((S_TOT,), jnp.int32),
            pltpu.VMEM_SHARED((S_TOT,), jnp.float32),
        ],
    )(_dispatch_body)
    return f(ids_flat, w_flat)


def _gather_body(stok_hbm, x_hbm, na_hbm, xs_hbm, idx_v, rows_v, nav, sem):
    w = lax.axis_index("s") * 2 + lax.axis_index("c")
    pltpu.sync_copy(na_hbm, nav)
    n_rows = nav[...][0] * BT
    for j in range(NGCH // NW + 1):
        cid = w + j * NW
        base = cid * GCH

        @pl.when((cid < NGCH) & (base < n_rows))
        def _():
            pltpu.sync_copy(stok_hbm.at[pl.ds(base, GCH)], idx_v)
            pltpu.async_copy(x_hbm.at[idx_v], rows_v, sem).wait()
            pltpu.sync_copy(rows_v, xs_hbm.at[pl.ds(base, GCH)])


def _gather(stok, x, na):
    mesh = plsc.VectorSubcoreMesh(core_axis_name="c", subcore_axis_name="s")
    f = functools.partial(
        pl.kernel,
        mesh=mesh,
        compiler_params=pltpu.CompilerParams(needs_layout_passes=False),
        out_type=jax.ShapeDtypeStruct((S, D), jnp.float32),
        scratch_types=[
            pltpu.VMEM((GCH,), jnp.int32),
            pltpu.VMEM((GCH, D), jnp.float32),
            pltpu.VMEM((16,), jnp.int32),
            pltpu.SemaphoreType.DMA,
        ],
    )(_gather_body)
    return f(stok, x, na)


def _gmm_body(be_ref, na_ref, sw_ref, xs_ref, eg_ref, eu_ref, ed_ref,
              out_ref, acc):
    b = pl.program_id(0)
    it = pl.program_id(1)
    na = na_ref[0]

    @pl.when(b < na)
    def _():
        @pl.when(it == 0)
        def _():
            acc[...] = jnp.zeros_like(acc)

        x = xs_ref[...]
        g = lax.dot_general(x, eg_ref[0], (((1,), (1,)), ((), ())),
                            preferred_element_type=jnp.float32)
        u = lax.dot_general(x, eu_ref[0], (((1,), (1,)), ((), ())),
                            preferred_element_type=jnp.float32)
        h = g * jax.nn.sigmoid(g) * u
        acc[...] += lax.dot_general(h, ed_ref[0], (((1,), (1,)), ((), ())),
                                    preferred_element_type=jnp.float32)

        @pl.when(it == IT - 1)
        def _():
            out_ref[...] = acc[...] * sw_ref[0, 0, :][:, None]


def _gmm(be, na, sw3, xs, eg, eu, ed):
    def bclamp(b, na_ref):
        return jnp.minimum(b, na_ref[0] - 1)

    grid_spec = pltpu.PrefetchScalarGridSpec(
        num_scalar_prefetch=2,
        grid=(NB, IT),
        in_specs=[
            pl.BlockSpec((1, 1, BT),
                         lambda b, it, be_r, na_r: (bclamp(b, na_r), 0, 0)),
            pl.BlockSpec((BT, D),
                         lambda b, it, be_r, na_r: (bclamp(b, na_r), 0)),
            pl.BlockSpec((1, TI, D),
                         lambda b, it, be_r, na_r:
                         (be_r[bclamp(b, na_r)],
                          jnp.where(b < na_r[0], it, IT - 1), 0)),
            pl.BlockSpec((1, TI, D),
                         lambda b, it, be_r, na_r:
                         (be_r[bclamp(b, na_r)],
                          jnp.where(b < na_r[0], it, IT - 1), 0)),
            pl.BlockSpec((1, D, TI),
                         lambda b, it, be_r, na_r:
                         (be_r[bclamp(b, na_r)], 0,
                          jnp.where(b < na_r[0], it, IT - 1))),
        ],
        out_specs=pl.BlockSpec((BT, D), lambda b, it, be_r, na_r: (b, 0)),
        scratch_shapes=[pltpu.VMEM((BT, D), jnp.float32)],
    )
    return pl.pallas_call(
        _gmm_body,
        grid_spec=grid_spec,
        out_shape=jax.ShapeDtypeStruct((S, D), jnp.float32),
    )(be, na, sw3, xs, eg, eu, ed)


def _combine_body(inv_hbm, ys_hbm, out_hbm, iv64, idx0, idx1, r0, r1, sem):
    w = lax.axis_index("s") * 2 + lax.axis_index("c")
    iota = lax.iota(jnp.int32, 16)
    for c in range(TPW // CCH):
        tbase = w * TPW + c * CCH
        pltpu.sync_copy(inv_hbm.at[pl.ds(2 * tbase, 2 * CCH)], iv64)
        for r in range(CCH // 16):
            lanes = (r * 16 + iota) * 2
            idx0[pl.ds(r * 16, 16)] = plsc.load_gather(iv64, [lanes])
            idx1[pl.ds(r * 16, 16)] = plsc.load_gather(iv64, [lanes + 1])
        pltpu.async_copy(ys_hbm.at[idx0], r0, sem).wait()
        pltpu.async_copy(ys_hbm.at[idx1], r1, sem).wait()

        def orow(rr, _):
            for cc in range(D // 16):
                sl = pl.ds(cc * 16, 16)
                r0[rr, sl] = r0[rr, sl] + r1[rr, sl]
            return 0

        lax.fori_loop(0, CCH, orow, 0)
        pltpu.sync_copy(r0, out_hbm.at[pl.ds(tbase, CCH)])


def _combine(inv, ys):
    mesh = plsc.VectorSubcoreMesh(core_axis_name="c", subcore_axis_name="s")
    f = functools.partial(
        pl.kernel,
        mesh=mesh,
        compiler_params=pltpu.CompilerParams(needs_layout_passes=False),
        out_type=jax.ShapeDtypeStruct((T, D), jnp.float32),
        scratch_types=[
            pltpu.VMEM((2 * CCH,), jnp.int32),
            pltpu.VMEM((CCH,), jnp.int32),
            pltpu.VMEM((CCH,), jnp.int32),
            pltpu.VMEM((CCH, D), jnp.float32),
            pltpu.VMEM((CCH, D), jnp.float32),
            pltpu.SemaphoreType.DMA,
        ],
    )(_combine_body)
    return f(inv, ys)


def kernel(x, gate_w, expert_gate, expert_up, expert_down):
    x_flat = x.reshape(-1, x.shape[-1])
    topk_ids, topk_w, aux = _router(x_flat, gate_w)
    stok, sw, inv, be, na = _dispatch(topk_ids.reshape(A),
                                      topk_w.reshape(A))
    xs = _gather(stok, x_flat, na)
    sw3 = sw[:S].reshape(NB, 1, BT)
    ys = _gmm(be, na, sw3, xs, expert_gate, expert_up, expert_down)
    out = _combine(inv, ys)
    return out.reshape(x.shape), aux[0, 0]
